# baseline (speedup 1.0000x reference)
import functools

import jax
import jax.numpy as jnp
from jax import lax
from jax.experimental import pallas as pl
from jax.experimental.pallas import tpu as pltpu
from jax.experimental.pallas import tpu_sc as plsc

_NC = 2
_NS = 16
_NW = _NC * _NS
_L = 16


@jax.jit
def _emb_lookup(xp, t_flat):
    npair, batch = xp.shape
    nrow = t_flat.shape[0] // _L
    blocks = npair // 4

    mesh = plsc.VectorSubcoreMesh(core_axis_name="c", subcore_axis_name="s")

    @functools.partial(
        pl.kernel,
        out_type=jax.ShapeDtypeStruct((npair, _L, batch), jnp.float32),
        mesh=mesh,
        scratch_types=[
            pltpu.VMEM((_L * nrow,), jnp.float32),
            pltpu.VMEM((2, 4, batch), jnp.int32),
            pltpu.VMEM((2, 2, _L, batch), jnp.float32),
            pltpu.SemaphoreType.DMA((2,)),
            pltpu.SemaphoreType.DMA((2,)),
        ],
        compiler_params=pltpu.CompilerParams(needs_layout_passes=False),
    )
    def k(xp_hbm, tab_hbm, out_hbm, tab_v, idx_v, obuf, sem_i, sem_o):
        wid = lax.axis_index("s") * _NC + lax.axis_index("c")
        pltpu.sync_copy(tab_hbm, tab_v)
        nb = blocks // _NW

        def idx_copy(i, slot):
            bid = wid + i * _NW
            return pltpu.make_async_copy(
                xp_hbm.at[pl.ds(bid * 4, 4)], idx_v.at[slot], sem_i.at[slot]
            )

        idx_copy(0, 0).start()

        def blk(i, carry):
            slot = lax.rem(i, 2)
            bid = wid + i * _NW

            @pl.when(i + 1 < nb)
            def _():
                idx_copy(i + 1, 1 - slot).start()

            idx_copy(i, slot).wait()

            for half in range(2):
                o = half
                p = bid * 4 + 2 * half

                @pl.when(i > 0)
                def _():
                    pltpu.make_async_copy(
                        obuf.at[o], out_hbm.at[pl.ds(p, 2)], sem_o.at[o]
                    ).wait()

                for h in range(2):
                    kk = 2 * half + h

                    @plsc.parallel_loop(0, batch // _L, unroll=2)
                    def bbody(bb):
                        v = idx_v[slot, kk, pl.ds(bb * _L, _L)]
                        vals = [
                            plsc.load_gather(tab_v, [v + j * nrow])
                            for j in range(_L)
                        ]
                        for j in range(_L):
                            obuf[o, h, j, pl.ds(bb * _L, _L)] = vals[j]

                pltpu.async_copy(
                    obuf.at[o], out_hbm.at[pl.ds(p, 2)], sem_o.at[o]
                )
            return carry

        lax.fori_loop(0, nb, blk, 0)

        last = (nb - 1) * _NW * 4 + wid * 4
        for o in range(2):
            pltpu.make_async_copy(
                obuf.at[o], out_hbm.at[pl.ds(last + 2 * o, 2)], sem_o.at[o]
            ).wait()

    return k(xp, t_flat)


def kernel(x, table):
    b, l, d = x.shape
    nrow, emb = table.shape
    xp = x.astype(jnp.int32).transpose(1, 2, 0).reshape(l * d, b)
    t_flat = table.T.reshape(emb * nrow)
    out = _emb_lookup(xp, t_flat)
    return out.reshape(l, d, emb, b).transpose(3, 0, 1, 2)


# speedup vs baseline: 1.0174x; 1.0174x over previous
import functools

import jax
import jax.numpy as jnp
from jax import lax
from jax.experimental import pallas as pl
from jax.experimental.pallas import tpu as pltpu
from jax.experimental.pallas import tpu_sc as plsc

_NC = 2
_NS = 16
_NW = _NC * _NS
_L = 16


@jax.jit
def _emb_lookup(xp, t_flat):
    npair, batch = xp.shape
    nrow = t_flat.shape[0] // _L
    blocks = npair // 4

    mesh = plsc.VectorSubcoreMesh(core_axis_name="c", subcore_axis_name="s")

    @functools.partial(
        pl.kernel,
        out_type=jax.ShapeDtypeStruct((npair, _L, batch), jnp.float32),
        mesh=mesh,
        scratch_types=[
            pltpu.VMEM((_L * nrow,), jnp.float32),
            pltpu.VMEM((2, 4, batch), jnp.int32),
            pltpu.VMEM((4, _L, batch), jnp.float32),
            pltpu.SemaphoreType.DMA((2,)),
            pltpu.SemaphoreType.DMA((4,)),
        ],
        compiler_params=pltpu.CompilerParams(needs_layout_passes=False),
    )
    def k(xp_hbm, tab_hbm, out_hbm, tab_v, idx_v, obuf, sem_i, sem_o):
        wid = lax.axis_index("s") * _NC + lax.axis_index("c")
        pltpu.sync_copy(tab_hbm, tab_v)
        nb = blocks // _NW

        def idx_copy(i, slot):
            bid = wid + i * _NW
            return pltpu.make_async_copy(
                xp_hbm.at[pl.ds(bid * 4, 4)], idx_v.at[slot], sem_i.at[slot]
            )

        idx_copy(0, 0).start()

        def blk(i, carry):
            slot = lax.rem(i, 2)
            bid = wid + i * _NW

            @pl.when(i + 1 < nb)
            def _():
                idx_copy(i + 1, 1 - slot).start()

            idx_copy(i, slot).wait()

            for kk in range(4):
                o = kk
                p = bid * 4 + kk

                @pl.when(i > 0)
                def _():
                    pltpu.make_async_copy(
                        obuf.at[o], out_hbm.at[p], sem_o.at[o]
                    ).wait()

                @plsc.parallel_loop(0, batch // _L, unroll=2)
                def bbody(bb):
                    v = idx_v[slot, kk, pl.ds(bb * _L, _L)]
                    vals = [
                        plsc.load_gather(tab_v, [v + j * nrow])
                        for j in range(_L)
                    ]
                    for j in range(_L):
                        obuf[o, j, pl.ds(bb * _L, _L)] = vals[j]
                pltpu.async_copy(obuf.at[o], out_hbm.at[p], sem_o.at[o])
            return carry

        lax.fori_loop(0, nb, blk, 0)

        last = (nb - 1) * _NW * 4 + wid * 4
        for o in range(4):
            pltpu.make_async_copy(
                obuf.at[o], out_hbm.at[last + o], sem_o.at[o]
            ).wait()

    return k(xp, t_flat)


def kernel(x, table):
    b, l, d = x.shape
    nrow, emb = table.shape
    xp = x.astype(jnp.int32).transpose(1, 2, 0).reshape(l * d, b)
    t_flat = table.T.reshape(emb * nrow)
    out = _emb_lookup(xp, t_flat)
    return out.reshape(l, d, emb, b).transpose(3, 0, 1, 2)
